# Initial kernel scaffold; baseline (speedup 1.0000x reference)
import jax
import jax.numpy as jnp
from jax.experimental import pallas as pl


def kernel(inputs, embeddings):
    raise NotImplementedError("write your pallas kernel here")


# speedup vs baseline: 1.4641x; 1.4641x over previous
import functools

import jax
import jax.numpy as jnp
from jax import lax
from jax.experimental import pallas as pl
from jax.experimental.pallas import tpu as pltpu
from jax.experimental.pallas import tpu_sc as plsc

_DIM = 32
_CODES = 512
_BM = 2048


def _argmin_body(x_ref, emb_ref, idx_ref, loss_ref):
    i = pl.program_id(0)
    x = x_ref[...]
    e = emb_ref[...]
    rowsq = jnp.sum(x * x, axis=1, keepdims=True)
    esq = jnp.sum(e * e, axis=0, keepdims=True)
    dot = jnp.dot(x, e, preferred_element_type=jnp.float32)
    dist = (rowsq - 2.0 * dot) + esq
    minval = jnp.min(dist, axis=1, keepdims=True)
    cols = lax.broadcasted_iota(jnp.int32, dist.shape, 1)
    idx = jnp.min(jnp.where(dist == minval, cols, _CODES), axis=1)
    idx_ref[...] = idx.astype(jnp.int32)

    @pl.when(i == 0)
    def _init():
        loss_ref[...] = jnp.zeros((1, 1), jnp.float32)

    loss_ref[...] += jnp.sum(minval).reshape(1, 1)


def _tc_argmin(flat_x, embeddings):
    n = flat_x.shape[0]
    grid = n // _BM
    return pl.pallas_call(
        _argmin_body,
        grid=(grid,),
        in_specs=[
            pl.BlockSpec((_BM, _DIM), lambda i: (i, 0)),
            pl.BlockSpec((_DIM, _CODES), lambda i: (0, 0)),
        ],
        out_specs=[
            pl.BlockSpec((_BM,), lambda i: (i,)),
            pl.BlockSpec((1, 1), lambda i: (0, 0)),
        ],
        out_shape=[
            jax.ShapeDtypeStruct((n,), jnp.int32),
            jax.ShapeDtypeStruct((1, 1), jnp.float32),
        ],
    )(flat_x, embeddings)


_SC_CORES = 2
_SC_SUBCORES = 16


def _make_sc_gather(n_tokens):
    nw = _SC_CORES * _SC_SUBCORES
    rows_per_w = n_tokens // nw
    n_idx_rows = rows_per_w // 128
    group = 8
    mesh = plsc.VectorSubcoreMesh(
        core_axis_name="c", subcore_axis_name="s",
        num_cores=_SC_CORES, num_subcores=_SC_SUBCORES)

    @functools.partial(
        pl.kernel,
        mesh=mesh,
        out_type=jax.ShapeDtypeStruct((n_tokens, _DIM), jnp.float32),
        scratch_types=[
            pltpu.VMEM((n_idx_rows, 128), jnp.int32),
            pltpu.VMEM((group * 128, _DIM), jnp.float32),
            pltpu.SemaphoreType.DMA,
        ],
        compiler_params=pltpu.CompilerParams(use_tc_tiling_on_sc=False),
    )
    def gather_kernel(table_hbm, idx_hbm, out_hbm, idx_v, rows_v, sem):
        wid = lax.axis_index("s") * _SC_CORES + lax.axis_index("c")
        base_row = wid * n_idx_rows
        pltpu.sync_copy(idx_hbm.at[pl.ds(base_row, n_idx_rows)], idx_v)
        for g in range(n_idx_rows // group):
            copies = []
            for t in range(group):
                copies.append(pltpu.async_copy(
                    table_hbm.at[idx_v.at[g * group + t]],
                    rows_v.at[pl.ds(t * 128, 128)],
                    sem,
                ))
            for c in copies:
                c.wait()
            out_off = wid * rows_per_w + g * group * 128
            pltpu.sync_copy(rows_v, out_hbm.at[pl.ds(out_off, group * 128)])

    return gather_kernel


def kernel(inputs, embeddings):
    in_shape = inputs.shape
    flat_x = inputs.reshape(-1, _DIM)
    n = flat_x.shape[0]
    idx, loss_sum = _tc_argmin(flat_x, embeddings)
    table = embeddings.T
    quant_flat = _make_sc_gather(n)(table, idx.reshape(n // 128, 128))
    loss = loss_sum[0, 0] * (1.25 / inputs.size)
    return quant_flat.reshape(in_shape), loss


# speedup vs baseline: 2.1408x; 1.4622x over previous
import functools

import jax
import jax.numpy as jnp
from jax import lax
from jax.experimental import pallas as pl
from jax.experimental.pallas import tpu as pltpu
from jax.experimental.pallas import tpu_sc as plsc

_DIM = 32
_CODES = 512
_BM = 2048


def _argmin_body(n_tokens, x_ref, emb_ref, idx_ref, loss_ref, tab_ref):
    i = pl.program_id(0)
    x = x_ref[...]
    e = emb_ref[...]
    rowsq = jnp.sum(x * x, axis=1, keepdims=True)
    esq = jnp.sum(e * e, axis=0, keepdims=True)
    dot = jnp.dot(x, e, preferred_element_type=jnp.float32)
    dist = (rowsq - 2.0 * dot) + esq
    minval = jnp.min(dist, axis=1, keepdims=True)
    cols = lax.broadcasted_iota(jnp.int32, (1, _CODES), 1).astype(jnp.float32)
    idxf = jnp.min(jnp.where(dist == minval, cols, float(_CODES)), axis=1)
    idx_ref[...] = idxf.astype(jnp.int32).reshape(_BM // 128, 128)
    part = jnp.dot(jnp.ones((1, _BM), jnp.float32), minval,
                   preferred_element_type=jnp.float32)

    @pl.when(i == 0)
    def _init():
        loss_ref[...] = jnp.zeros((1, 1), jnp.float32)
        tab_ref[...] = e.T

    loss_ref[...] += part

    @pl.when(i == pl.num_programs(0) - 1)
    def _scale():
        loss_ref[...] *= 1.25 / (n_tokens * _DIM)


def _tc_argmin(flat_x, embeddings):
    n = flat_x.shape[0]
    grid = n // _BM
    return pl.pallas_call(
        functools.partial(_argmin_body, n),
        grid=(grid,),
        in_specs=[
            pl.BlockSpec((_BM, _DIM), lambda i: (i, 0)),
            pl.BlockSpec((_DIM, _CODES), lambda i: (0, 0)),
        ],
        out_specs=[
            pl.BlockSpec((_BM // 128, 128), lambda i: (i, 0)),
            pl.BlockSpec((1, 1), lambda i: (0, 0)),
            pl.BlockSpec((_CODES, _DIM), lambda i: (0, 0)),
        ],
        out_shape=[
            jax.ShapeDtypeStruct((n // 128, 128), jnp.int32),
            jax.ShapeDtypeStruct((1, 1), jnp.float32),
            jax.ShapeDtypeStruct((_CODES, _DIM), jnp.float32),
        ],
    )(flat_x, embeddings)


_SC_CORES = 2
_SC_SUBCORES = 16


def _make_sc_gather(n_tokens):
    nw = _SC_CORES * _SC_SUBCORES
    rows_per_w = n_tokens // nw
    n_idx_rows = rows_per_w // 128
    group = 8
    mesh = plsc.VectorSubcoreMesh(
        core_axis_name="c", subcore_axis_name="s",
        num_cores=_SC_CORES, num_subcores=_SC_SUBCORES)

    @functools.partial(
        pl.kernel,
        mesh=mesh,
        out_type=jax.ShapeDtypeStruct((n_tokens, _DIM), jnp.float32),
        scratch_types=[
            pltpu.VMEM((n_idx_rows, 128), jnp.int32),
            pltpu.VMEM((group * 128, _DIM), jnp.float32),
            pltpu.SemaphoreType.DMA,
        ],
        compiler_params=pltpu.CompilerParams(use_tc_tiling_on_sc=False),
    )
    def gather_kernel(table_hbm, idx_hbm, out_hbm, idx_v, rows_v, sem):
        wid = lax.axis_index("s") * _SC_CORES + lax.axis_index("c")
        base_row = wid * n_idx_rows
        pltpu.sync_copy(idx_hbm.at[pl.ds(base_row, n_idx_rows)], idx_v)
        for g in range(n_idx_rows // group):
            copies = []
            for t in range(group):
                copies.append(pltpu.async_copy(
                    table_hbm.at[idx_v.at[g * group + t]],
                    rows_v.at[pl.ds(t * 128, 128)],
                    sem,
                ))
            for c in copies:
                c.wait()
            out_off = wid * rows_per_w + g * group * 128
            pltpu.sync_copy(rows_v, out_hbm.at[pl.ds(out_off, group * 128)])

    return gather_kernel


def kernel(inputs, embeddings):
    in_shape = inputs.shape
    flat_x = inputs.reshape(-1, _DIM)
    n = flat_x.shape[0]
    idx2d, loss, table = _tc_argmin(flat_x, embeddings)
    quant_flat = _make_sc_gather(n)(table, idx2d)
    return quant_flat.reshape(in_shape), loss[0, 0]
